# baseline (speedup 1.0000x reference)
import functools

import jax
import jax.numpy as jnp
from jax import lax
from jax.experimental import pallas as pl
from jax.experimental.pallas import tpu as pltpu

NB = 8
CH = 512


def _pipeline_kernel(te_hbm, pos_hbm, out_hbm, te_bufs, out_bufs, pos_buf,
                     lsem, ssem, psem, *, B, S, E):
    cpb = S // CH
    total = B * cpb

    def load_te(c, slot):
        b = c // cpb
        r = c - b * cpb
        off = pl.multiple_of(r * CH, CH)
        return pltpu.make_async_copy(
            te_hbm.at[b, pl.ds(off, CH)], te_bufs.at[slot], lsem.at[slot])

    def load_pos(c, slot):
        off = pl.multiple_of(c * CH, CH)
        return pltpu.make_async_copy(
            pos_hbm.at[pl.ds(off, CH)], pos_buf.at[pl.ds(off, CH)],
            psem.at[slot])

    def store_out(c, slot):
        b = c // cpb
        r = c - b * cpb
        off = pl.multiple_of(r * CH, CH)
        return pltpu.make_async_copy(
            out_bufs.at[slot], out_hbm.at[b, pl.ds(off, CH)], ssem.at[slot])

    for slot in range(NB):
        load_te(slot, slot).start()
        load_pos(slot, slot).start()

    def group(g, carry):
        for slot in range(NB):
            c = g * NB + slot
            load_te(c, slot).wait()

            @pl.when(c < cpb)
            def _():
                load_pos(c, slot).wait()

            @pl.when(c >= NB)
            def _():
                store_out(c - NB, slot).wait()

            r = c - (c // cpb) * cpb
            off = pl.multiple_of(r * CH, CH)
            out_bufs[slot] = te_bufs[slot] + pos_buf[pl.ds(off, CH), :]
            store_out(c, slot).start()

            nxt = c + NB

            @pl.when(nxt < total)
            def _():
                load_te(nxt, slot).start()

            @pl.when(nxt < cpb)
            def _():
                load_pos(nxt, slot).start()
        return carry

    lax.fori_loop(0, total // NB, group, 0)

    for slot in range(NB):
        pltpu.make_async_copy(
            out_bufs.at[slot], out_hbm.at[0, pl.ds(0, CH)],
            ssem.at[slot]).wait()


def kernel(token_embedding, pos_table):
    B, S, E = token_embedding.shape
    return pl.pallas_call(
        functools.partial(_pipeline_kernel, B=B, S=S, E=E),
        in_specs=[
            pl.BlockSpec(memory_space=pl.ANY),
            pl.BlockSpec(memory_space=pl.ANY),
        ],
        out_specs=pl.BlockSpec(memory_space=pl.ANY),
        out_shape=jax.ShapeDtypeStruct((B, S, E), token_embedding.dtype),
        scratch_shapes=[
            pltpu.VMEM((NB, CH, E), jnp.float32),
            pltpu.VMEM((NB, CH, E), jnp.float32),
            pltpu.VMEM((S, E), jnp.float32),
            pltpu.SemaphoreType.DMA((NB,)),
            pltpu.SemaphoreType.DMA((NB,)),
            pltpu.SemaphoreType.DMA((NB,)),
        ],
    )(token_embedding, pos_table)
